# Initial kernel scaffold; baseline (speedup 1.0000x reference)
import jax
import jax.numpy as jnp
from jax.experimental import pallas as pl


def kernel(inputs, user_table, item_table):
    raise NotImplementedError("write your pallas kernel here")


# speedup vs baseline: 1.0357x; 1.0357x over previous
import functools

import jax
import jax.numpy as jnp
from jax import lax
from jax.experimental import pallas as pl
from jax.experimental.pallas import tpu as pltpu
from jax.experimental.pallas import tpu_sc as plsc

LANES = 16


def _sc_kernel_body(num_chunks, chunk, d, uid_hbm, iid_hbm, ut_hbm, it_hbm,
                    out_hbm, idx_u, idx_i, ubuf, ibuf, dots, outv, sem_u,
                    sem_i):
    nc = 2
    wid = lax.axis_index("s") * nc + lax.axis_index("c")
    per_w = num_chunks * chunk
    base = wid * per_w

    d_steps = d // LANES

    for j in range(num_chunks):
        pltpu.sync_copy(uid_hbm.at[pl.ds(base + j * chunk, chunk)],
                        idx_u.at[j])
        pltpu.sync_copy(iid_hbm.at[pl.ds(base + j * chunk, chunk)],
                        idx_i.at[j])
        cu = pltpu.async_copy(ut_hbm.at[idx_u.at[j]], ubuf, sem_u)
        ci = pltpu.async_copy(it_hbm.at[idx_i.at[j]], ibuf, sem_i)
        cu.wait()
        ci.wait()

        def group_body(g, carry, j=j):
            for l in range(LANES):
                p = g * LANES + l
                part = ubuf[p, pl.ds(0, LANES)] * ibuf[p, pl.ds(0, LANES)]
                for k in range(1, d_steps):
                    part = part + (ubuf[p, pl.ds(k * LANES, LANES)] *
                                   ibuf[p, pl.ds(k * LANES, LANES)])
                dots[l, pl.ds(0, LANES)] = part
            lane = lax.iota(jnp.int32, LANES)
            score = plsc.load_gather(dots, [lane, jnp.zeros((LANES,), jnp.int32)])
            for col in range(1, LANES):
                score = score + plsc.load_gather(
                    dots, [lane, jnp.full((LANES,), col, jnp.int32)])
            rating = 1.0 + 4.0 / (1.0 + jnp.exp(-score))
            outv[pl.ds(j * chunk + g * LANES, LANES)] = rating
            return carry

        lax.fori_loop(0, chunk // LANES, group_body, 0)

    pltpu.sync_copy(outv, out_hbm.at[pl.ds(base, per_w)])


def _forward(uid, iid, user_table, item_table):
    b = uid.shape[0]
    d = user_table.shape[1]
    nw = 32
    per_w = b // nw
    chunk = 128
    num_chunks = per_w // chunk

    mesh = plsc.VectorSubcoreMesh(core_axis_name="c", subcore_axis_name="s")
    kfn = pl.kernel(
        functools.partial(_sc_kernel_body, num_chunks, chunk, d),
        mesh=mesh,
        compiler_params=pltpu.CompilerParams(needs_layout_passes=False),
        out_type=jax.ShapeDtypeStruct((b,), jnp.float32),
        scratch_types=[
            pltpu.VMEM((num_chunks, chunk), jnp.int32),
            pltpu.VMEM((num_chunks, chunk), jnp.int32),
            pltpu.VMEM((chunk, d), jnp.float32),
            pltpu.VMEM((chunk, d), jnp.float32),
            pltpu.VMEM((LANES, LANES), jnp.float32),
            pltpu.VMEM((per_w,), jnp.float32),
            pltpu.SemaphoreType.DMA,
            pltpu.SemaphoreType.DMA,
        ],
    )
    return kfn(uid, iid, user_table, item_table)


def kernel(inputs, user_table, item_table):
    uid = inputs[:, 0].astype(jnp.int32)
    iid = inputs[:, 1].astype(jnp.int32)
    return _forward(uid, iid, user_table, item_table)


# speedup vs baseline: 1.0759x; 1.0388x over previous
import functools

import jax
import jax.numpy as jnp
from jax import lax
from jax.experimental import pallas as pl
from jax.experimental.pallas import tpu as pltpu
from jax.experimental.pallas import tpu_sc as plsc

LANES = 16


def _sc_kernel_body(num_chunks, chunk, d, inp_hbm, ut_hbm, it_hbm, out_hbm,
                    pairs, idx_u, idx_i, ubuf, ibuf, dots, outv,
                    sem_u0, sem_u1, sem_i0, sem_i1):
    nc = 2
    wid = lax.axis_index("s") * nc + lax.axis_index("c")
    per_w = num_chunks * chunk
    base = wid * per_w
    d_steps = d // LANES

    pltpu.sync_copy(inp_hbm.at[pl.ds(base, per_w)], pairs)
    lane = lax.iota(jnp.int32, LANES)
    zero = jnp.zeros((LANES,), jnp.int32)
    one = jnp.ones((LANES,), jnp.int32)
    for t in range(per_w // LANES):
        row = t * LANES + lane
        idx_u[t * LANES // chunk, pl.ds(t * LANES % chunk, LANES)] = (
            plsc.load_gather(pairs, [row, zero]))
        idx_i[t * LANES // chunk, pl.ds(t * LANES % chunk, LANES)] = (
            plsc.load_gather(pairs, [row, one]))

    sems_u = (sem_u0, sem_u1)
    sems_i = (sem_i0, sem_i1)

    def start(j):
        s = j % 2
        cu = pltpu.async_copy(ut_hbm.at[idx_u.at[j]], ubuf.at[s], sems_u[s])
        ci = pltpu.async_copy(it_hbm.at[idx_i.at[j]], ibuf.at[s], sems_i[s])
        return cu, ci

    pending = start(0)
    for j in range(num_chunks):
        cu, ci = pending
        if j + 1 < num_chunks:
            nxt = start(j + 1)
        cu.wait()
        ci.wait()
        if j + 1 < num_chunks:
            pending = nxt
        s = j % 2

        def group_body(g, carry, j=j, s=s):
            for l in range(LANES):
                p = g * LANES + l
                part = (ubuf[s, p, pl.ds(0, LANES)] *
                        ibuf[s, p, pl.ds(0, LANES)])
                for k in range(1, d_steps):
                    part = part + (ubuf[s, p, pl.ds(k * LANES, LANES)] *
                                   ibuf[s, p, pl.ds(k * LANES, LANES)])
                dots[l, pl.ds(0, LANES)] = part
            score = plsc.load_gather(dots, [lane, zero])
            for col in range(1, LANES):
                score = score + plsc.load_gather(
                    dots, [lane, jnp.full((LANES,), col, jnp.int32)])
            rating = 1.0 + 4.0 / (1.0 + jnp.exp(-score))
            outv[pl.ds(j * chunk + g * LANES, LANES)] = rating
            return carry

        lax.fori_loop(0, chunk // LANES, group_body, 0)

    pltpu.sync_copy(outv, out_hbm.at[pl.ds(base, per_w)])


def _forward(inputs, user_table, item_table):
    b = inputs.shape[0]
    d = user_table.shape[1]
    nw = 32
    per_w = b // nw
    chunk = 64
    num_chunks = per_w // chunk

    mesh = plsc.VectorSubcoreMesh(core_axis_name="c", subcore_axis_name="s")
    kfn = pl.kernel(
        functools.partial(_sc_kernel_body, num_chunks, chunk, d),
        mesh=mesh,
        compiler_params=pltpu.CompilerParams(needs_layout_passes=False),
        out_type=jax.ShapeDtypeStruct((b,), jnp.float32),
        scratch_types=[
            pltpu.VMEM((per_w, 2), jnp.int32),
            pltpu.VMEM((num_chunks, chunk), jnp.int32),
            pltpu.VMEM((num_chunks, chunk), jnp.int32),
            pltpu.VMEM((2, chunk, d), jnp.float32),
            pltpu.VMEM((2, chunk, d), jnp.float32),
            pltpu.VMEM((LANES, LANES), jnp.float32),
            pltpu.VMEM((per_w,), jnp.float32),
            pltpu.SemaphoreType.DMA,
            pltpu.SemaphoreType.DMA,
            pltpu.SemaphoreType.DMA,
            pltpu.SemaphoreType.DMA,
        ],
    )
    return kfn(inputs, user_table, item_table)


def kernel(inputs, user_table, item_table):
    return _forward(inputs.astype(jnp.int32), user_table, item_table)


# speedup vs baseline: 1.3860x; 1.2882x over previous
import functools

import jax
import jax.numpy as jnp
from jax import lax
from jax.experimental import pallas as pl
from jax.experimental.pallas import tpu as pltpu
from jax.experimental.pallas import tpu_sc as plsc

LANES = 16


def _sc_kernel_body(num_chunks, chunk, d, uid_hbm, iid_hbm, ut_hbm, it_hbm,
                    out_hbm, idx_u, idx_i, ubuf, ibuf, dots, outv,
                    sem_u0, sem_u1, sem_i0, sem_i1):
    nc = 2
    wid = lax.axis_index("s") * nc + lax.axis_index("c")
    per_w = num_chunks * chunk
    base = wid * per_w
    d_steps = d // LANES

    pltpu.sync_copy(uid_hbm.at[pl.ds(base, per_w)], idx_u)
    pltpu.sync_copy(iid_hbm.at[pl.ds(base, per_w)], idx_i)

    sems_u = (sem_u0, sem_u1)
    sems_i = (sem_i0, sem_i1)

    def start(j):
        s = j % 2
        cu = pltpu.async_copy(ut_hbm.at[idx_u.at[pl.ds(j * chunk, chunk)]],
                              ubuf.at[s], sems_u[s])
        ci = pltpu.async_copy(it_hbm.at[idx_i.at[pl.ds(j * chunk, chunk)]],
                              ibuf.at[s], sems_i[s])
        return cu, ci

    lane = lax.iota(jnp.int32, LANES)
    pending = start(0)
    for j in range(num_chunks):
        cu, ci = pending
        if j + 1 < num_chunks:
            nxt = start(j + 1)
        cu.wait()
        ci.wait()
        if j + 1 < num_chunks:
            pending = nxt
        s = j % 2

        @plsc.parallel_loop(0, chunk, unroll=4)
        def pair_body(p, s=s):
            m = [ubuf[s, p, pl.ds(k * LANES, LANES)] *
                 ibuf[s, p, pl.ds(k * LANES, LANES)] for k in range(d_steps)]
            while len(m) > 1:
                m = [m[2 * t] + m[2 * t + 1] for t in range(len(m) // 2)]
            dots[p, pl.ds(0, LANES)] = m[0]

        @plsc.parallel_loop(0, chunk // LANES)
        def group_body(g, j=j):
            row = g * LANES + lane
            score = plsc.load_gather(dots, [row, jnp.zeros((LANES,), jnp.int32)])
            for col in range(1, LANES):
                score = score + plsc.load_gather(
                    dots, [row, jnp.full((LANES,), col, jnp.int32)])
            rating = 1.0 + 4.0 / (1.0 + jnp.exp(-score))
            outv[pl.ds(j * chunk + g * LANES, LANES)] = rating

    pltpu.sync_copy(outv, out_hbm.at[pl.ds(base, per_w)])


def _forward(uid, iid, user_table, item_table):
    b = uid.shape[0]
    d = user_table.shape[1]
    nw = 32
    per_w = b // nw
    chunk = 128
    num_chunks = per_w // chunk

    mesh = plsc.VectorSubcoreMesh(core_axis_name="c", subcore_axis_name="s")
    kfn = pl.kernel(
        functools.partial(_sc_kernel_body, num_chunks, chunk, d),
        mesh=mesh,
        compiler_params=pltpu.CompilerParams(needs_layout_passes=False),
        out_type=jax.ShapeDtypeStruct((b,), jnp.float32),
        scratch_types=[
            pltpu.VMEM((per_w,), jnp.int32),
            pltpu.VMEM((per_w,), jnp.int32),
            pltpu.VMEM((2, chunk, d), jnp.float32),
            pltpu.VMEM((2, chunk, d), jnp.float32),
            pltpu.VMEM((chunk, LANES), jnp.float32),
            pltpu.VMEM((per_w,), jnp.float32),
            pltpu.SemaphoreType.DMA,
            pltpu.SemaphoreType.DMA,
            pltpu.SemaphoreType.DMA,
            pltpu.SemaphoreType.DMA,
        ],
    )
    return kfn(uid, iid, user_table, item_table)


def kernel(inputs, user_table, item_table):
    ids = inputs.astype(jnp.int32)
    return _forward(ids[:, 0], ids[:, 1], user_table, item_table)
